# baseline (speedup 1.0000x reference)
import functools

import jax
import jax.numpy as jnp
from jax import lax
from jax.experimental import pallas as pl
from jax.experimental.pallas import tpu as pltpu
from jax.experimental.pallas import tpu_sc as plsc

B = 4
L = 8192
D = 768
CHUNK = 64


@functools.cache
def _build_sc_kernel():
    info = plsc.get_sparse_core_info()
    nw = info.num_cores * info.num_subcores
    rows_per_w = L // nw
    n_chunks = rows_per_w // CHUNK

    mesh = plsc.VectorSubcoreMesh(core_axis_name="c", subcore_axis_name="s")

    @functools.partial(
        pl.kernel,
        mesh=mesh,
        out_type=jax.ShapeDtypeStruct((B, L, D), jnp.float32),
        scratch_types=[
            pltpu.VMEM((2, CHUNK, D), jnp.float32),
            pltpu.SemaphoreType.DMA,
            pltpu.SemaphoreType.DMA((2,)),
        ],
    )
    def k(emb_hbm, out_hbm, buf, lsem, ssem):
        wid = lax.axis_index("s") * info.num_cores + lax.axis_index("c")
        base = wid * rows_per_w

        def load(c):
            cp = pltpu.make_async_copy(
                emb_hbm.at[pl.ds(base + c * CHUNK, CHUNK)], buf.at[c % 2], lsem
            )
            cp.start()
            return cp

        def stores(c):
            row = base + c * CHUNK
            cps = [
                pltpu.make_async_copy(
                    buf.at[c % 2], out_hbm.at[b, pl.ds(row, CHUNK)], ssem.at[c % 2]
                )
                for b in range(B)
            ]
            for cp in cps:
                cp.start()
            return cps

        pending_load = load(0)
        pending_stores = [None, None]
        for c in range(n_chunks):
            pending_load.wait()
            new_stores = stores(c)
            if pending_stores[(c + 1) % 2] is not None:
                for cp in pending_stores[(c + 1) % 2]:
                    cp.wait()
                pending_stores[(c + 1) % 2] = None
            pending_stores[c % 2] = new_stores
            if c + 1 < n_chunks:
                pending_load = load(c + 1)
        for slot in (0, 1):
            if pending_stores[slot] is not None:
                for cp in pending_stores[slot]:
                    cp.wait()

    return k


def kernel(x, Embed):
    return _build_sc_kernel()(Embed)
